# baseline (speedup 1.0000x reference)
import functools

import jax
import jax.numpy as jnp
from jax import lax
from jax.experimental import pallas as pl
from jax.experimental.pallas import tpu as pltpu
from jax.experimental.pallas import tpu_sc as plsc

_NUM_EMBEDDINGS = 1000000
_D = 32
_B = 16384
_H = 50
_NW = 32
_HSPLIT = 25
_GROUPS = (_B // 128) * _HSPLIT
_GPW = _GROUPS // _NW
_GPC = 4
_CHUNK = _GPC * 128
_NCHUNK = _GPW // _GPC


def _emb_body(idx_hbm, table_hbm, out_raw, idx_v, rows_v, gsem, wsem):
    wid = lax.axis_index("s") * 2 + lax.axis_index("c")
    g0 = wid * _GPW

    def chunk_pos(c):
        g = g0 + c * _GPC
        return g // 128, (g % 128) * 128

    def fire_chunk(c, slot):
        h, off = chunk_pos(c)
        pltpu.sync_copy(idx_hbm.at[h, pl.ds(off, _CHUNK)], idx_v.at[slot])
        for j in range(_GPC):
            pltpu.async_copy(
                table_hbm.at[idx_v.at[slot, pl.ds(j * 128, 128)]],
                rows_v.at[slot, pl.ds(j * 128, 128)],
                gsem.at[slot],
            )

    def drain_gathers(slot):
        pltpu.make_async_copy(
            table_hbm.at[pl.ds(0, _CHUNK)], rows_v.at[slot], gsem.at[slot]
        ).wait()

    def out_dst(c):
        h, off = chunk_pos(c)
        return out_raw.at[h, pl.ds(off, _CHUNK)]

    fire_chunk(0, 0)

    def body(c, _):
        slot = lax.rem(c, 2)
        nslot = 1 - slot

        @pl.when(c + 1 < _NCHUNK)
        def _prefetch():
            @pl.when(c >= 1)
            def _drain_write():
                pltpu.make_async_copy(
                    rows_v.at[nslot], out_dst(c - 1), wsem.at[nslot]
                ).wait()

            fire_chunk(c + 1, nslot)

        drain_gathers(slot)
        pltpu.async_copy(rows_v.at[slot], out_dst(c), wsem.at[slot])
        return _

    lax.fori_loop(0, _NCHUNK, body, None)

    last = _NCHUNK - 1
    pltpu.make_async_copy(
        rows_v.at[lax.rem(last, 2)], out_dst(last), wsem.at[lax.rem(last, 2)]
    ).wait()

    @pl.when(_NCHUNK >= 2)
    def _():
        pltpu.make_async_copy(
            rows_v.at[lax.rem(last - 1, 2)],
            out_dst(last - 1),
            wsem.at[lax.rem(last - 1, 2)],
        ).wait()


@jax.jit
def _emb(token_ids_t4, table):
    mesh = plsc.VectorSubcoreMesh(core_axis_name="c", subcore_axis_name="s")
    run = functools.partial(
        pl.kernel,
        mesh=mesh,
        out_type=jax.ShapeDtypeStruct((_HSPLIT, _B, _D), jnp.float32),
        scratch_types=[
            pltpu.VMEM((2, _CHUNK), jnp.int32),
            pltpu.VMEM((2, _CHUNK, _D), jnp.float32),
            pltpu.SemaphoreType.DMA((2,)),
            pltpu.SemaphoreType.DMA((2,)),
        ],
        compiler_params=pltpu.CompilerParams(use_tc_tiling_on_sc=False),
    )(_emb_body)
    return run(token_ids_t4, table)


def kernel(token_ids, weight):
    table = jnp.pad(weight, ((0, 0), (0, 128 - _D))).reshape(-1, _D)
    idx4 = (token_ids.astype(jnp.int32) * 4).T
    y1 = _emb(idx4[:_HSPLIT], table)
    y2 = _emb(idx4[_HSPLIT:], table)
    return jnp.concatenate(
        [y1.transpose(1, 0, 2), y2.transpose(1, 0, 2)], axis=1
    )


# speedup vs baseline: 1.0354x; 1.0354x over previous
import functools

import jax
import jax.numpy as jnp
from jax import lax
from jax.experimental import pallas as pl
from jax.experimental.pallas import tpu as pltpu
from jax.experimental.pallas import tpu_sc as plsc

_NUM_EMBEDDINGS = 1000000
_D = 32
_B = 16384
_H = 50
_NW = 32
_GROUPS = (_B // 128) * _H
_GPW = _GROUPS // _NW
_GPC = 8
_CHUNK = _GPC * 128
_NCHUNK = _GPW // _GPC


def _emb_body(idx_hbm, table_hbm, out_raw, idx_v, rows_v, gsem, wsem):
    wid = lax.axis_index("s") * 2 + lax.axis_index("c")
    g0 = wid * _GPW

    def chunk_pos(c):
        g = g0 + c * _GPC
        return g // 128, (g % 128) * 128

    def fire_chunk(c, slot):
        h, off = chunk_pos(c)
        pltpu.sync_copy(idx_hbm.at[h, pl.ds(off, _CHUNK)], idx_v.at[slot])
        for j in range(_GPC):
            pltpu.async_copy(
                table_hbm.at[idx_v.at[slot, pl.ds(j * 128, 128)]],
                rows_v.at[slot, pl.ds(j * 128, 128)],
                gsem.at[slot],
            )

    def drain_gathers(slot):
        pltpu.make_async_copy(
            table_hbm.at[pl.ds(0, _CHUNK)], rows_v.at[slot], gsem.at[slot]
        ).wait()

    def out_dst(c):
        h, off = chunk_pos(c)
        return out_raw.at[h, pl.ds(off, _CHUNK)]

    fire_chunk(0, 0)

    def body(c, _):
        slot = lax.rem(c, 2)
        nslot = 1 - slot

        @pl.when(c + 1 < _NCHUNK)
        def _prefetch():
            @pl.when(c >= 1)
            def _drain_write():
                pltpu.make_async_copy(
                    rows_v.at[nslot], out_dst(c - 1), wsem.at[nslot]
                ).wait()

            fire_chunk(c + 1, nslot)

        drain_gathers(slot)
        pltpu.async_copy(rows_v.at[slot], out_dst(c), wsem.at[slot])
        return _

    lax.fori_loop(0, _NCHUNK, body, None)

    last = _NCHUNK - 1
    pltpu.make_async_copy(
        rows_v.at[lax.rem(last, 2)], out_dst(last), wsem.at[lax.rem(last, 2)]
    ).wait()

    @pl.when(_NCHUNK >= 2)
    def _():
        pltpu.make_async_copy(
            rows_v.at[lax.rem(last - 1, 2)],
            out_dst(last - 1),
            wsem.at[lax.rem(last - 1, 2)],
        ).wait()


@jax.jit
def _emb(token_ids_t4, table):
    mesh = plsc.VectorSubcoreMesh(core_axis_name="c", subcore_axis_name="s")
    run = functools.partial(
        pl.kernel,
        mesh=mesh,
        out_type=jax.ShapeDtypeStruct((_H, _B, _D), jnp.float32),
        scratch_types=[
            pltpu.VMEM((2, _CHUNK), jnp.int32),
            pltpu.VMEM((2, _CHUNK, _D), jnp.float32),
            pltpu.SemaphoreType.DMA((2,)),
            pltpu.SemaphoreType.DMA((2,)),
        ],
        compiler_params=pltpu.CompilerParams(use_tc_tiling_on_sc=False),
    )(_emb_body)
    return run(token_ids_t4, table)


def kernel(token_ids, weight):
    table = jnp.pad(weight, ((0, 0), (0, 128 - _D))).reshape(-1, _D)
    idx4 = (token_ids.astype(jnp.int32) * 4).T
    out_t = _emb(idx4, table)
    return out_t.transpose(1, 0, 2)
